# baseline (speedup 1.0000x reference)
import functools

import jax
import jax.numpy as jnp
from jax import lax
from jax.experimental import pallas as pl
from jax.experimental.pallas import tpu as pltpu
from jax.experimental.pallas import tpu_sc as plsc

_V = 1000
_NTOK = 16 * 2048
_NC = 2
_NS = 16
_L = 16
_NW = _NC * _NS
_BPW = _NTOK // _NW
_VP = 1024
_CH = 32
_NCHUNK = _BPW // _CH


def _lse_body(table_ref, out_ref):
    x = table_ref[...]
    m = jnp.max(x, axis=1, keepdims=True)
    s = jnp.sum(jnp.exp(x - m), axis=1, keepdims=True)
    out_ref[...] = m + jnp.log(s)


def _row_lse(table):
    return pl.pallas_call(
        _lse_body,
        out_shape=jax.ShapeDtypeStruct((_V, 1), jnp.float32),
    )(table)


@functools.partial(
    pl.kernel,
    out_type=(
        jax.ShapeDtypeStruct((_NTOK, _V), jnp.float32),
    ),
    mesh=plsc.VectorSubcoreMesh(core_axis_name="c", subcore_axis_name="s"),
    compiler_params=pltpu.CompilerParams(use_tc_tiling_on_sc=False),
    scratch_types=(
        pltpu.VMEM((_CH,), jnp.int32),
        pltpu.VMEM((_CH, _V), jnp.float32),
        pltpu.SemaphoreType.DMA,
    ),
)
def _sc_lookup(table, idxf, out, idx_v, rows_v, sem):
    c = lax.axis_index("c")
    s = lax.axis_index("s")
    wid = s * _NC + c
    base = wid * _BPW

    for n in range(_NCHUNK):
        off = base + n * _CH
        pltpu.sync_copy(idxf.at[pl.ds(off, _CH)], idx_v)
        pltpu.async_copy(table.at[idx_v], rows_v, sem).wait()
        pltpu.sync_copy(rows_v, out.at[pl.ds(off, _CH)])


def kernel(idx, targets, table):
    idxf = idx.reshape(-1).astype(jnp.int32)
    tgtf = targets.reshape(-1).astype(jnp.int32)
    table = table.astype(jnp.float32)
    lse = _row_lse(table).reshape(_V)
    (logits2,) = _sc_lookup(table, idxf)
    nll = lse[idxf] - table[idxf, tgtf]
    loss = jnp.mean(nll)
    return (logits2, loss)


# speedup vs baseline: 1.2814x; 1.2814x over previous
import functools

import jax
import jax.numpy as jnp
from jax import lax
from jax.experimental import pallas as pl
from jax.experimental.pallas import tpu as pltpu
from jax.experimental.pallas import tpu_sc as plsc

_V = 1000
_NTOK = 16 * 2048
_NC = 2
_NS = 16
_L = 16
_NW = _NC * _NS
_BPW = _NTOK // _NW
_CH = 64
_NCHUNK = _BPW // _CH


def _lse_body(table_ref, out_ref):
    x = table_ref[...]
    m = jnp.max(x, axis=1, keepdims=True)
    s = jnp.sum(jnp.exp(x - m), axis=1, keepdims=True)
    out_ref[...] = m + jnp.log(s)


def _row_lse(table):
    return pl.pallas_call(
        _lse_body,
        out_shape=jax.ShapeDtypeStruct((_V, 1), jnp.float32),
    )(table)


@functools.partial(
    pl.kernel,
    out_type=(
        jax.ShapeDtypeStruct((_NTOK, _V), jnp.float32),
        jax.ShapeDtypeStruct((_NW, _L), jnp.float32),
    ),
    mesh=plsc.VectorSubcoreMesh(core_axis_name="c", subcore_axis_name="s"),
    compiler_params=pltpu.CompilerParams(use_tc_tiling_on_sc=False),
    scratch_types=(
        pltpu.VMEM((_CH,), jnp.float32),
        pltpu.VMEM((_CH,), jnp.int32),
        pltpu.VMEM((_CH,), jnp.int32),
        pltpu.VMEM((_CH,), jnp.int32),
        pltpu.VMEM((_CH, _V), jnp.float32),
        pltpu.VMEM((_CH, _V), jnp.float32),
        pltpu.VMEM((_L,), jnp.float32),
        pltpu.VMEM((_CH,), jnp.int32),
        pltpu.VMEM((_CH,), jnp.float32),
        pltpu.SemaphoreType.DMA,
        pltpu.SemaphoreType.DMA,
        pltpu.SemaphoreType.DMA,
        pltpu.SemaphoreType.DMA,
    ),
)
def _sc_lookup(table, tflat, idxf, tgtf, lse, out, part,
               lval_v, idx_a, idx_b, tgt_v, rows_a, rows_b, acc_v, lin_v, tval_v,
               gsem_a, gsem_b, ssem_a, ssem_b):
    c = lax.axis_index("c")
    s = lax.axis_index("s")
    wid = s * _NC + c
    base = wid * _BPW

    bufs = ((idx_a, rows_a, gsem_a, ssem_a), (idx_b, rows_b, gsem_b, ssem_b))

    def gather(n):
        idx_v, rows_v, gsem, _ = bufs[n % 2]
        return pltpu.make_async_copy(table.at[idx_v], rows_v, gsem)

    def scatter(n):
        _, rows_v, _, ssem = bufs[n % 2]
        return pltpu.make_async_copy(
            rows_v, out.at[pl.ds(base + n * _CH, _CH)], ssem)

    pltpu.sync_copy(idxf.at[pl.ds(base, _CH)], idx_a)
    gather(0).start()

    acc = jnp.zeros((_L,), jnp.float32)
    lane = lax.iota(jnp.int32, _L)
    for n in range(_NCHUNK):
        idx_v, rows_v, _, _ = bufs[n % 2]
        off = base + n * _CH
        if n + 1 < _NCHUNK:
            if n >= 1:
                scatter(n - 1).wait()
            nidx_v = bufs[(n + 1) % 2][0]
            pltpu.sync_copy(idxf.at[pl.ds(off + _CH, _CH)], nidx_v)
            gather(n + 1).start()
        gather(n).wait()
        pltpu.sync_copy(tgtf.at[pl.ds(off, _CH)], tgt_v)
        for j in range(_CH // _L):
            tg = tgt_v[pl.ds(j * _L, _L)]
            iv = idx_v[pl.ds(j * _L, _L)]
            lin_v[pl.ds(j * _L, _L)] = iv * _V + tg
        pltpu.async_copy(tflat.at[lin_v], tval_v, ssem_a).wait()
        pltpu.async_copy(lse.at[idx_v], lval_v, ssem_b).wait()
        for j in range(_CH // _L):
            tv = tval_v[pl.ds(j * _L, _L)]
            lv = lval_v[pl.ds(j * _L, _L)]
            acc = acc + (lv - tv)
        scatter(n).start()

    scatter(_NCHUNK - 2).wait()
    scatter(_NCHUNK - 1).wait()

    acc_v[...] = acc
    pltpu.sync_copy(acc_v, part.at[wid])


def kernel(idx, targets, table):
    idxf = idx.reshape(-1).astype(jnp.int32)
    tgtf = targets.reshape(-1).astype(jnp.int32)
    table = table.astype(jnp.float32)
    lse = _row_lse(table).reshape(_V)
    tflat = jnp.pad(table.reshape(-1), (0, 8))
    logits2, part = _sc_lookup(table, tflat, idxf, tgtf, lse)
    loss = jnp.sum(part) / _NTOK
    return (logits2, loss)


# speedup vs baseline: 1.2929x; 1.0090x over previous
import functools

import jax
import jax.numpy as jnp
from jax import lax
from jax.experimental import pallas as pl
from jax.experimental.pallas import tpu as pltpu
from jax.experimental.pallas import tpu_sc as plsc

_V = 1000
_NTOK = 16 * 2048
_NC = 2
_NS = 16
_L = 16
_NW = _NC * _NS
_BPW = _NTOK // _NW
_CH = 32
_NCHUNK = _BPW // _CH
_NBUF = 3


def _lse_body(table_ref, out_ref):
    x = table_ref[...]
    m = jnp.max(x, axis=1, keepdims=True)
    s = jnp.sum(jnp.exp(x - m), axis=1, keepdims=True)
    out_ref[...] = m + jnp.log(s)


def _row_lse(table):
    return pl.pallas_call(
        _lse_body,
        out_shape=jax.ShapeDtypeStruct((_V, 1), jnp.float32),
    )(table)


@functools.partial(
    pl.kernel,
    out_type=(
        jax.ShapeDtypeStruct((_NTOK, _V), jnp.float32),
        jax.ShapeDtypeStruct((_NW, _L), jnp.float32),
    ),
    mesh=plsc.VectorSubcoreMesh(core_axis_name="c", subcore_axis_name="s"),
    compiler_params=pltpu.CompilerParams(use_tc_tiling_on_sc=False),
    scratch_types=(
        pltpu.VMEM((_BPW,), jnp.int32),
        pltpu.VMEM((_BPW,), jnp.int32),
        pltpu.VMEM((_BPW,), jnp.float32),
        pltpu.VMEM((_BPW,), jnp.float32),
        pltpu.VMEM((_CH, _V), jnp.float32),
        pltpu.VMEM((_CH, _V), jnp.float32),
        pltpu.VMEM((_CH, _V), jnp.float32),
        pltpu.SemaphoreType.DMA,
        pltpu.SemaphoreType.DMA,
        pltpu.SemaphoreType.DMA,
        pltpu.SemaphoreType.DMA,
        pltpu.SemaphoreType.DMA,
        pltpu.SemaphoreType.DMA,
        pltpu.SemaphoreType.DMA,
    ),
)
def _sc_lookup(table, tflat, idxf, tgtf, lse, out, part,
               idx_v, lin_v, tval_v, lval_v, rows_0, rows_1, rows_2,
               gsem_0, gsem_1, gsem_2, ssem_0, ssem_1, ssem_2, lsem):
    c = lax.axis_index("c")
    s = lax.axis_index("s")
    wid = s * _NC + c
    base = wid * _BPW

    rows = (rows_0, rows_1, rows_2)
    gsem = (gsem_0, gsem_1, gsem_2)
    ssem = (ssem_0, ssem_1, ssem_2)

    def gather(n):
        b = n % _NBUF
        return pltpu.make_async_copy(
            table.at[idx_v.at[pl.ds(n * _CH, _CH)]], rows[b], gsem[b])

    def scatter(n):
        b = n % _NBUF
        return pltpu.make_async_copy(
            rows[b], out.at[pl.ds(base + n * _CH, _CH)], ssem[b])

    pltpu.sync_copy(idxf.at[pl.ds(base, _BPW)], idx_v)
    pltpu.sync_copy(tgtf.at[pl.ds(base, _BPW)], lin_v)
    gather(0).start()
    gather(1).start()

    for n in range(_NCHUNK):
        if n + 2 < _NCHUNK:
            if n >= 1:
                scatter(n - 1).wait()
            gather(n + 2).start()
        gather(n).wait()
        scatter(n).start()

    for j in range(_BPW // _L):
        sl = pl.ds(j * _L, _L)
        lin_v[sl] = idx_v[sl] * _V + lin_v[sl]
    pltpu.async_copy(tflat.at[lin_v], tval_v, lsem).wait()
    pltpu.async_copy(lse.at[idx_v], lval_v, lsem).wait()
    acc = jnp.zeros((_L,), jnp.float32)
    for j in range(_BPW // _L):
        sl = pl.ds(j * _L, _L)
        acc = acc + (lval_v[sl] - tval_v[sl])
    tval_v[pl.ds(0, _L)] = acc
    pltpu.sync_copy(tval_v.at[pl.ds(0, _L)], part.at[wid])

    scatter(_NCHUNK - 2).wait()
    scatter(_NCHUNK - 1).wait()


def kernel(idx, targets, table):
    idxf = idx.reshape(-1).astype(jnp.int32)
    tgtf = targets.reshape(-1).astype(jnp.int32)
    table = table.astype(jnp.float32)
    lse = _row_lse(table).reshape(_V)
    tflat = jnp.pad(table.reshape(-1), (0, 8))
    logits2, part = _sc_lookup(table, tflat, idxf, tgtf, lse)
    loss = jnp.sum(part) / _NTOK
    return (logits2, loss)
